# baseline (speedup 1.0000x reference)
import functools

import jax
import jax.numpy as jnp
from jax import lax
from jax.experimental import pallas as pl
from jax.experimental.pallas import tpu as pltpu
from jax.experimental.pallas import tpu_sc as plsc


def kernel(x, token_table, pos_table):
    B, S = x.shape
    V, D = token_table.shape
    assert pos_table.shape == (S, D)

    info = plsc.get_sparse_core_info()
    NC, NS = info.num_cores, info.num_subcores
    NW = NC * NS
    N = B // NW
    H = S // 2
    HP = ((H + 15) // 16) * 16
    NSTEP = 2 * N
    NB = 4
    G = NSTEP // NB

    mesh = plsc.VectorSubcoreMesh(core_axis_name="c", subcore_axis_name="s")

    @functools.partial(
        pl.kernel,
        mesh=mesh,
        out_type=jax.ShapeDtypeStruct((B, 2, H * D), jnp.float32),
        scratch_types=[
            pltpu.VMEM((2, H * D), jnp.float32),
            pltpu.VMEM((NB, HP), jnp.int32),
            pltpu.VMEM((NB, H * D), jnp.float32),
            pltpu.SemaphoreType.DMA((NB,)),
            pltpu.SemaphoreType.DMA((NB,)),
            pltpu.SemaphoreType.DMA((NB,)),
        ],
        compiler_params=pltpu.CompilerParams(use_tc_tiling_on_sc=False),
    )
    def tpe(x_hbm, tok_hbm, pos_hbm, out_hbm, pos_v, raw_v, row_v,
            gsem, ism, ssem):
        wid = lax.axis_index("s") * NC + lax.axis_index("c")
        base = wid * NSTEP
        last = base + NSTEP - 1
        pltpu.sync_copy(pos_hbm, pos_v)

        def issue_rows(slot):
            def blk(b, toks):
                tv = raw_v[slot, pl.ds(b * 16, 16)]
                for l in range(toks):
                    pltpu.make_async_copy(
                        tok_hbm.at[tv[l]],
                        row_v.at[slot, pl.ds((b * 16 + l) * D, D)],
                        gsem.at[slot]).start()

            def blk_body(b, cr):
                blk(b, 16)
                return cr

            lax.fori_loop(0, H // 16, blk_body, 0)
            if H % 16:
                blk(H // 16, H % 16)

        def gwait(slot):
            pltpu.make_async_copy(
                out_hbm.at[0, 0], row_v.at[slot], gsem.at[slot]).wait()

        def raw_load(chunk, slot):
            c = jnp.minimum(chunk, last)
            return pltpu.make_async_copy(
                x_hbm.at[c // 2, c % 2], raw_v.at[slot], ism.at[slot])

        def store(chunk, slot):
            return pltpu.make_async_copy(
                row_v.at[slot], out_hbm.at[chunk // 2, chunk % 2],
                ssem.at[slot])

        for j in range(2):
            pltpu.sync_copy(x_hbm.at[(base + j) // 2, (base + j) % 2],
                            raw_v.at[j])
            issue_rows(j)
        for j in range(2, NB):
            raw_load(base + j, j).start()

        def body(g, carry):
            for k in range(NB):
                c = base + g * NB + k
                half = k % 2
                k2 = (k + 2) % NB

                gwait(k)

                raw_load(c + 2, k2).wait()
                @pl.when(c - 2 >= base)
                def _():
                    store(c - 2, k2).wait()
                @pl.when(c + 2 <= last)
                def _():
                    issue_rows(k2)

                def add_row(s_, cr):
                    for j in range(D // 16):
                        sl = pl.ds(s_ * D + j * 16, 16)
                        row_v[k, sl] = row_v[k, sl] + pos_v[half, sl]
                    return cr

                lax.fori_loop(0, H, add_row, 0, unroll=2)

                raw_load(c + NB, k).start()

                store(c, k).start()
            return carry

        lax.fori_loop(0, G, body, 0)

        for k in range(2, NB):
            store(base + NSTEP - NB + k + 2, k).wait()
            raw_load(last, k).wait()

    x3 = jnp.pad(x.reshape(B, 2, H).astype(jnp.int32),
                 ((0, 0), (0, 0), (0, HP - H)))
    out = tpe(x3, token_table, pos_table.reshape(2, H * D))
    return out.reshape(B, S, D)


# speedup vs baseline: 1.0954x; 1.0954x over previous
import functools

import jax
import jax.numpy as jnp
from jax import lax
from jax.experimental import pallas as pl
from jax.experimental.pallas import tpu as pltpu
from jax.experimental.pallas import tpu_sc as plsc


def kernel(x, token_table, pos_table):
    B, S = x.shape
    V, D = token_table.shape
    assert pos_table.shape == (S, D)

    info = plsc.get_sparse_core_info()
    NC, NS = info.num_cores, info.num_subcores
    NW = NC * NS
    N = B // NW
    H = S // 2
    HP = ((H + 15) // 16) * 16
    NSTEP = 2 * N
    NB = 4
    G = NSTEP // NB

    mesh = plsc.VectorSubcoreMesh(core_axis_name="c", subcore_axis_name="s")

    @functools.partial(
        pl.kernel,
        mesh=mesh,
        out_type=jax.ShapeDtypeStruct((B, 2, H, D), jnp.float32),
        scratch_types=[
            pltpu.VMEM((2, H, D), jnp.float32),
            pltpu.VMEM((NB, HP), jnp.int32),
            pltpu.VMEM((NB, HP, D), jnp.float32),
            pltpu.SemaphoreType.DMA((NB,)),
            pltpu.SemaphoreType.DMA((NB,)),
            pltpu.SemaphoreType.DMA((NB,)),
        ],
        compiler_params=pltpu.CompilerParams(use_tc_tiling_on_sc=False),
    )
    def tpe(x_hbm, tok_hbm, pos_hbm, out_hbm, pos_v, raw_v, row_v,
            gsem, ism, ssem):
        wid = lax.axis_index("s") * NC + lax.axis_index("c")
        base = wid * NSTEP
        last = base + NSTEP - 1
        pltpu.sync_copy(pos_hbm, pos_v)

        def issue_rows(slot):
            def blk(start):
                tv = raw_v[slot, pl.ds(start, 16)]
                pltpu.make_async_copy(
                    tok_hbm.at[tv],
                    row_v.at[slot, pl.ds(start, 16)],
                    gsem.at[slot]).start()

            def blk_body(b, cr):
                blk(b * 16)
                return cr

            lax.fori_loop(0, H // 16, blk_body, 0)
            if H % 16:
                blk(HP - 16)

        def gwait(slot):
            pltpu.make_async_copy(
                tok_hbm.at[pl.ds(0, HP)], row_v.at[slot], gsem.at[slot]).wait()

        def raw_load(chunk, slot):
            c = jnp.minimum(chunk, last)
            return pltpu.make_async_copy(
                x_hbm.at[c // 2, c % 2], raw_v.at[slot], ism.at[slot])

        def store(chunk, slot):
            return pltpu.make_async_copy(
                row_v.at[slot, pl.ds(0, H)], out_hbm.at[chunk // 2, chunk % 2],
                ssem.at[slot])

        for j in range(2):
            pltpu.sync_copy(x_hbm.at[(base + j) // 2, (base + j) % 2],
                            raw_v.at[j])
            issue_rows(j)
        for j in range(2, NB):
            raw_load(base + j, j).start()

        def body(g, carry):
            for k in range(NB):
                c = base + g * NB + k
                half = k % 2
                k2 = (k + 2) % NB

                gwait(k)

                raw_load(c + 2, k2).wait()
                @pl.when(c - 2 >= base)
                def _():
                    store(c - 2, k2).wait()
                @pl.when(c + 2 <= last)
                def _():
                    issue_rows(k2)

                def add_row(s_, cr):
                    for j in range(D // 16):
                        sl = pl.ds(j * 16, 16)
                        row_v[k, s_, sl] = row_v[k, s_, sl] + pos_v[half, s_, sl]
                    return cr

                lax.fori_loop(0, H, add_row, 0, unroll=2)

                raw_load(c + NB, k).start()

                store(c, k).start()
            return carry

        lax.fori_loop(0, G, body, 0)

        for k in range(2, NB):
            store(base + NSTEP - NB + k + 2, k).wait()
            raw_load(last, k).wait()

    x3 = jnp.pad(x.reshape(B, 2, H).astype(jnp.int32),
                 ((0, 0), (0, 0), (0, HP - H)), mode="edge")
    out = tpe(x3, token_table, pos_table.reshape(2, H, D))
    return out.reshape(B, S, D)


# speedup vs baseline: 1.2865x; 1.1745x over previous
import functools

import jax
import jax.numpy as jnp
from jax import lax
from jax.experimental import pallas as pl
from jax.experimental.pallas import tpu as pltpu
from jax.experimental.pallas import tpu_sc as plsc


def kernel(x, token_table, pos_table):
    B, S = x.shape
    V, D = token_table.shape
    assert pos_table.shape == (S, D)

    info = plsc.get_sparse_core_info()
    NC, NS = info.num_cores, info.num_subcores
    NW = NC * NS
    N = B // NW
    H = S // 2
    HP = 112
    NSTEP = 2 * N
    NB = 4
    G = NSTEP // NB
    W = 2 * D

    mesh = plsc.VectorSubcoreMesh(core_axis_name="c", subcore_axis_name="s")

    @functools.partial(
        pl.kernel,
        mesh=mesh,
        out_type=jax.ShapeDtypeStruct((B, 2, H // 2, W), jnp.float32),
        compiler_params=pltpu.CompilerParams(disable_bounds_checks=True),
        scratch_types=[
            pltpu.VMEM((2, H // 2, W), jnp.float32),
            pltpu.VMEM((NB, HP), jnp.int32),
            pltpu.VMEM((NB, HP), jnp.int32),
            pltpu.VMEM((NB, H, W), jnp.float32),
            pltpu.SemaphoreType.DMA((NB,)),
            pltpu.SemaphoreType.DMA((NB,)),
            pltpu.SemaphoreType.DMA((NB,)),
        ],
    )
    def tpe(x_hbm, tok_hbm, pos_hbm, out_hbm, pos_v, raw_v, hix_v,
            wide_v, gsem, ism, ssem):
        wid = lax.axis_index("s") * NC + lax.axis_index("c")
        base = wid * NSTEP
        last = base + NSTEP - 1
        pltpu.sync_copy(pos_hbm, pos_v)

        def gather(slot):
            return pltpu.make_async_copy(
                tok_hbm.at[hix_v.at[slot, pl.ds(0, H)]], wide_v.at[slot],
                gsem.at[slot])

        def raw_load(chunk, slot):
            c = jnp.minimum(chunk, last)
            return pltpu.make_async_copy(
                x_hbm.at[c // 2, c % 2], raw_v.at[slot, pl.ds(0, H)],
                ism.at[slot])

        def store(chunk, slot):
            return pltpu.make_async_copy(
                wide_v.at[slot, pl.ds(0, H // 2)],
                out_hbm.at[chunk // 2, chunk % 2], ssem.at[slot])

        def halve(slot):
            for b in range(HP // 16):
                sl = pl.ds(b * 16, 16)
                hix_v[slot, sl] = raw_v[slot, sl] >> 1

        for j in range(2):
            pltpu.sync_copy(x_hbm.at[(base + j) // 2, (base + j) % 2],
                            raw_v.at[j, pl.ds(0, H)])
            halve(j)
            gather(j).start()
        for j in range(2, NB):
            raw_load(base + j, j).start()

        def body(g, carry):
            for k in range(NB):
                c = base + g * NB + k
                half = k % 2
                k2 = (k + 2) % NB

                gather(k).wait()

                raw_load(c + 2, k2).wait()
                halve(k2)
                @pl.when(c - 2 >= base)
                def _():
                    store(c - 2, k2).wait()
                @pl.when(c + 2 <= last)
                def _():
                    gather(k2).start()

                def sel_tok(b, l, offv):
                    off = offv[l]
                    u = b * 8 + (l >> 1)
                    hi = (l & 1) * D
                    for j in range(D // 16):
                        sl = pl.ds(hi + j * 16, 16)
                        wide_v[k, u, sl] = (
                            wide_v[k, b * 16 + l, pl.ds(off + j * 16, 16)]
                            + pos_v[half, u, sl])

                def sel_blk(b, cr):
                    offv = (raw_v[k, pl.ds(b * 16, 16)] & 1) * D
                    for l in range(16):
                        sel_tok(b, l, offv)
                    return cr

                lax.fori_loop(0, H // 16, sel_blk, 0)
                offv_t = (raw_v[k, pl.ds((H // 16) * 16, 16)] & 1) * D
                for l in range(H % 16):
                    sel_tok(H // 16, l, offv_t)

                raw_load(c + NB, k).start()

                store(c, k).start()
            return carry

        lax.fori_loop(0, G, body, 0)

        for k in range(2, NB):
            store(base + NSTEP - NB + k + 2, k).wait()
            raw_load(last, k).wait()

    x3 = x.reshape(B, 2, H).astype(jnp.int32)
    out = tpe(x3, token_table.reshape(V // 2, W),
              pos_table.reshape(2, H // 2, W))
    return out.reshape(B, S, D)


# speedup vs baseline: 1.2872x; 1.0005x over previous
import functools

import jax
import jax.numpy as jnp
from jax import lax
from jax.experimental import pallas as pl
from jax.experimental.pallas import tpu as pltpu
from jax.experimental.pallas import tpu_sc as plsc


def kernel(x, token_table, pos_table):
    B, S = x.shape
    V, D = token_table.shape
    assert pos_table.shape == (S, D)

    info = plsc.get_sparse_core_info()
    NC, NS = info.num_cores, info.num_subcores
    NW = NC * NS
    N = B // NW
    H = S // 2
    HP = 112
    NSTEP = 2 * N
    NB = 4
    G = NSTEP // NB
    W = 2 * D

    mesh = plsc.VectorSubcoreMesh(core_axis_name="c", subcore_axis_name="s")

    @functools.partial(
        pl.kernel,
        mesh=mesh,
        out_type=jax.ShapeDtypeStruct((B, 2, H // 2, W), jnp.float32),
        scratch_types=[
            pltpu.VMEM((2, H // 2, W), jnp.float32),
            pltpu.VMEM((NB, HP), jnp.int32),
            pltpu.VMEM((NB, HP), jnp.int32),
            pltpu.VMEM((NB, H, W), jnp.float32),
            pltpu.SemaphoreType.DMA((NB,)),
            pltpu.SemaphoreType.DMA((NB,)),
            pltpu.SemaphoreType.DMA((NB,)),
        ],
    )
    def tpe(x_hbm, tok_hbm, pos_hbm, out_hbm, pos_v, raw_v, hix_v,
            wide_v, gsem, ism, ssem):
        wid = lax.axis_index("s") * NC + lax.axis_index("c")
        base = wid * NSTEP
        last = base + NSTEP - 1
        pltpu.sync_copy(pos_hbm, pos_v)

        def gather(slot):
            return pltpu.make_async_copy(
                tok_hbm.at[hix_v.at[slot, pl.ds(0, H)]], wide_v.at[slot],
                gsem.at[slot])

        def raw_load(chunk, slot):
            c = jnp.minimum(chunk, last)
            return pltpu.make_async_copy(
                x_hbm.at[c // 2, c % 2], raw_v.at[slot, pl.ds(0, H)],
                ism.at[slot])

        def store(chunk, slot):
            return pltpu.make_async_copy(
                wide_v.at[slot, pl.ds(0, H // 2)],
                out_hbm.at[chunk // 2, chunk % 2], ssem.at[slot])

        def halve(slot):
            for b in range(HP // 16):
                sl = pl.ds(b * 16, 16)
                hix_v[slot, sl] = raw_v[slot, sl] >> 1

        for j in range(2):
            pltpu.sync_copy(x_hbm.at[(base + j) // 2, (base + j) % 2],
                            raw_v.at[j, pl.ds(0, H)])
            halve(j)
            gather(j).start()
        for j in range(2, NB):
            raw_load(base + j, j).start()

        def body(g, carry):
            for k in range(NB):
                c = base + g * NB + k
                half = k % 2
                k2 = (k + 2) % NB

                gather(k).wait()

                raw_load(c + 2, k2).wait()
                halve(k2)
                @pl.when(c - 2 >= base)
                def _():
                    store(c - 2, k2).wait()
                @pl.when(c + 2 <= last)
                def _():
                    gather(k2).start()

                def sel_tok(b, l, offv):
                    off = offv[l]
                    u = b * 8 + (l >> 1)
                    hi = (l & 1) * D
                    for j in range(D // 16):
                        sl = pl.ds(hi + j * 16, 16)
                        wide_v[k, u, sl] = (
                            wide_v[k, b * 16 + l, pl.ds(off + j * 16, 16)]
                            + pos_v[half, u, sl])

                def sel_blk(b, cr):
                    offv = (raw_v[k, pl.ds(b * 16, 16)] & 1) * D
                    for l in range(16):
                        sel_tok(b, l, offv)
                    return cr

                lax.fori_loop(0, H // 16, sel_blk, 0)
                offv_t = (raw_v[k, pl.ds((H // 16) * 16, 16)] & 1) * D
                for l in range(H % 16):
                    sel_tok(H // 16, l, offv_t)

                raw_load(c + NB, k).start()

                store(c, k).start()
            return carry

        lax.fori_loop(0, G, body, 0)

        for k in range(2, NB):
            store(base + NSTEP - NB + k + 2, k).wait()
            raw_load(last, k).wait()

    x3 = x.reshape(B, 2, H).astype(jnp.int32)
    out = tpe(x3, token_table.reshape(V // 2, W),
              pos_table.reshape(2, H // 2, W))
    return out.reshape(B, S, D)
